# baseline (speedup 1.0000x reference)
import functools

import jax
import jax.numpy as jnp
from jax import lax
from jax.experimental import pallas as pl
from jax.experimental.pallas import tpu as pltpu
from jax.experimental.pallas import tpu_sc as plsc

NB = 64
BLK = 64
TOPK = 6
NW = 32
NG = 96
NEG = float("-inf")


def _scores_body(pq_ref, pk_ref, s_ref):
    s = lax.dot_general(
        pq_ref[...].astype(jnp.bfloat16), pk_ref[...].astype(jnp.bfloat16),
        (((2,), (2,)), ((0,), (0,))),
        preferred_element_type=jnp.float32)
    s_ref[...] = jnp.stack(
        [jnp.swapaxes(s[:, 16 * gi:16 * (gi + 1), :], 1, 2)
         for gi in range(4)], axis=1)


def _tree_max(vals):
    vals = list(vals)
    while len(vals) > 1:
        vals = [jnp.maximum(vals[i], vals[i + 1])
                for i in range(0, len(vals) - 1, 2)] + (
                    [vals[-1]] if len(vals) % 2 else [])
    return vals[0]


def _make_topk_kernel():
    mesh = plsc.VectorSubcoreMesh(core_axis_name="c", subcore_axis_name="s")

    @functools.partial(
        pl.kernel, mesh=mesh,
        out_type=jax.ShapeDtypeStruct((NG, NB, 16), jnp.float32),
        compiler_params=pltpu.CompilerParams(use_tc_tiling_on_sc=True),
        scratch_types=[
            pltpu.VMEM((NB, 16), jnp.float32),
            pltpu.VMEM((NB, 16), jnp.float32),
        ],
    )
    def topk_mask(scg_hbm, maskg_hbm, sc_v, mk_v):
        wid = lax.axis_index("s") * 2 + lax.axis_index("c")
        negv = jnp.full((16,), NEG, jnp.float32)
        onev = jnp.full((16,), 1.0, jnp.float32)
        zerov = jnp.full((16,), 0.0, jnp.float32)
        for t in range(NG // NW):
            g = wid + NW * t
            pltpu.sync_copy(scg_hbm.at[g], sc_v)
            cur = [sc_v[p, :] for p in range(NB)]
            for _ in range(TOPK - 1):
                thr = _tree_max(cur)
                cur = [jnp.where(c >= thr, negv, c) for c in cur]
            thr = _tree_max(cur)
            for p in range(NB):
                sel = (cur[p] >= thr) | (cur[p] == negv)
                mk_v[p, :] = jnp.where(sel, onev, zerov)
            pltpu.sync_copy(mk_v, maskg_hbm.at[g])

    return topk_mask


def _softmax_bf16(x):
    xb = x.astype(jnp.bfloat16).astype(jnp.float32)
    m = jnp.max(xb, axis=-1, keepdims=True)
    e = jnp.exp(xb - m)
    return e / jnp.sum(e, axis=-1, keepdims=True)


def _attn_body(q_ref, k_ref, v_ref, mask_ref, o_ref):
    qf = q_ref[0, 0]
    kf = k_ref[0, 0]
    vf = v_ref[0, 0]
    mg = mask_ref[0]
    mask = jnp.concatenate(
        [jnp.swapaxes(mg[gi], 0, 1) for gi in range(4)],
        axis=0)

    cq = _softmax_bf16(qf)
    ck = _softmax_bf16(kf)
    vb = vf.astype(jnp.bfloat16).astype(jnp.float32)

    ck3 = ck.reshape(NB, BLK, 64)
    vb3 = vb.reshape(NB, BLK, 64)

    M = lax.dot_general(
        ck3, vb3, (((1,), (1,)), ((0,), (0,))),
        preferred_element_type=jnp.float32)
    ksum = jnp.sum(ck3, axis=1)

    Msum = lax.dot_general(
        mask, M, (((1,), (0,)), ((), ())),
        preferred_element_type=jnp.float32)
    dsum = lax.dot_general(
        mask, ksum, (((1,), (0,)), ((), ())),
        preferred_element_type=jnp.float32)

    cq3 = cq.reshape(NB, BLK, 64)
    num = lax.dot_general(
        cq3, Msum, (((2,), (1,)), ((0,), (0,))),
        preferred_element_type=jnp.float32)
    den = lax.dot_general(
        cq3, dsum, (((2,), (1,)), ((0,), (0,))),
        preferred_element_type=jnp.float32)

    o = num / (den[..., None] + 1e-6)
    o_ref[0, 0] = o.reshape(NB * BLK, 64)


@jax.jit
def kernel(q, k, v):
    B, H, L, D = q.shape
    BH = B * H
    pq = q.reshape(B, H, NB, BLK, D).mean(axis=3).reshape(BH, NB, D)
    pk = k.reshape(B, H, NB, BLK, D).mean(axis=3).reshape(BH, NB, D)

    scg = pl.pallas_call(
        _scores_body,
        out_shape=jax.ShapeDtypeStruct((BH, 4, NB, 16), jnp.float32),
    )(pq, pk)

    maskg = _make_topk_kernel()(scg.reshape(NG, NB, 16))

    spec = pl.BlockSpec((1, 1, L, D), lambda b, h: (b, h, 0, 0))
    mspec = pl.BlockSpec((1, 4, NB, 16), lambda b, h: (b * H + h, 0, 0, 0))
    return pl.pallas_call(
        _attn_body,
        grid=(B, H),
        in_specs=[spec, spec, spec, mspec],
        out_specs=spec,
        out_shape=jax.ShapeDtypeStruct((B, H, L, D), jnp.float32),
    )(q, k, v, maskg.reshape(BH, 4, NB, 16))


# speedup vs baseline: 1.1451x; 1.1451x over previous
import functools

import jax
import jax.numpy as jnp
from jax import lax
from jax.experimental import pallas as pl
from jax.experimental.pallas import tpu as pltpu
from jax.experimental.pallas import tpu_sc as plsc

NB = 64
BLK = 64
TOPK = 6
NW = 32
NG = 96
NEG = float("-inf")


def _scores_body(pq_ref, pk_ref, s_ref):
    s = lax.dot_general(
        pq_ref[...].astype(jnp.bfloat16), pk_ref[...].astype(jnp.bfloat16),
        (((2,), (2,)), ((0,), (0,))),
        preferred_element_type=jnp.float32)
    s_ref[...] = jnp.stack(
        [jnp.swapaxes(s[:, 16 * gi:16 * (gi + 1), :], 1, 2)
         for gi in range(4)], axis=1)


def _tree_max(vals):
    vals = list(vals)
    while len(vals) > 1:
        vals = [jnp.maximum(vals[i], vals[i + 1])
                for i in range(0, len(vals) - 1, 2)] + (
                    [vals[-1]] if len(vals) % 2 else [])
    return vals[0]


def _make_topk_kernel():
    mesh = plsc.VectorSubcoreMesh(core_axis_name="c", subcore_axis_name="s")

    @functools.partial(
        pl.kernel, mesh=mesh,
        out_type=jax.ShapeDtypeStruct((NG, NB, 16), jnp.float32),
        compiler_params=pltpu.CompilerParams(use_tc_tiling_on_sc=True),
        scratch_types=[
            pltpu.VMEM((NB, 16), jnp.float32),
            pltpu.VMEM((NB, 16), jnp.float32),
        ],
    )
    def topk_mask(scg_hbm, maskg_hbm, sc_v, mk_v):
        wid = lax.axis_index("s") * 2 + lax.axis_index("c")
        negv = jnp.full((16,), NEG, jnp.float32)
        onev = jnp.full((16,), 1.0, jnp.float32)
        zerov = jnp.full((16,), 0.0, jnp.float32)
        for t in range(NG // NW):
            g = wid + NW * t
            pltpu.sync_copy(scg_hbm.at[g], sc_v)
            cur = [sc_v[p, :] for p in range(NB)]
            for _ in range(TOPK - 1):
                thr = _tree_max(cur)
                cur = [jnp.where(c >= thr, negv, c) for c in cur]
            thr = _tree_max(cur)
            for p in range(NB):
                sel = (cur[p] >= thr) | (cur[p] == negv)
                mk_v[p, :] = jnp.where(sel, onev, zerov)
            pltpu.sync_copy(mk_v, maskg_hbm.at[g])

    return topk_mask


def _softmax_bf16(x):
    xb = x.astype(jnp.float32)
    m = jnp.max(xb, axis=-1, keepdims=True)
    e = jnp.exp(xb - m)
    return e / jnp.sum(e, axis=-1, keepdims=True)


def _attn_body(q_ref, k_ref, v_ref, mask_ref, o_ref):
    qf = q_ref[0]
    kf = k_ref[0]
    vf = v_ref[0]
    mg = mask_ref[0]
    mask = jnp.concatenate(
        [jnp.swapaxes(mg[gi], 0, 1) for gi in range(4)],
        axis=0)

    cq = _softmax_bf16(qf)
    ck = _softmax_bf16(kf)
    vb = vf.astype(jnp.float32)

    ck3 = ck.reshape(NB, BLK, 64)
    vb3 = vb.reshape(NB, BLK, 64)

    M = lax.dot_general(
        ck3, vb3, (((1,), (1,)), ((0,), (0,))),
        preferred_element_type=jnp.float32)
    ksum = jnp.sum(ck3, axis=1)

    Msum = lax.dot_general(
        mask, M, (((1,), (0,)), ((), ())),
        preferred_element_type=jnp.float32)
    dsum = lax.dot_general(
        mask, ksum, (((1,), (0,)), ((), ())),
        preferred_element_type=jnp.float32)

    cq3 = cq.reshape(NB, BLK, 64)
    num = lax.dot_general(
        cq3, Msum, (((2,), (1,)), ((0,), (0,))),
        preferred_element_type=jnp.float32)
    den = lax.dot_general(
        cq3, dsum, (((2,), (1,)), ((0,), (0,))),
        preferred_element_type=jnp.float32)

    o = num / (den[..., None] + 1e-6)
    o_ref[0] = o.reshape(NB * BLK, 64)


@jax.jit
def kernel(q, k, v):
    B, H, L, D = q.shape
    BH = B * H
    pq = q.reshape(B, H, NB, BLK, D).mean(axis=3).reshape(BH, NB, D)
    pk = k.reshape(B, H, NB, BLK, D).mean(axis=3).reshape(BH, NB, D)

    scg = pl.pallas_call(
        _scores_body,
        out_shape=jax.ShapeDtypeStruct((BH, 4, NB, 16), jnp.float32),
    )(pq, pk)

    maskg = _make_topk_kernel()(scg.reshape(NG, NB, 16))

    qf = q.astype(jnp.bfloat16).reshape(BH, L, D)
    kf = k.astype(jnp.bfloat16).reshape(BH, L, D)
    vf = v.astype(jnp.bfloat16).reshape(BH, L, D)
    spec = pl.BlockSpec((1, L, D), lambda i: (i, 0, 0))
    mspec = pl.BlockSpec((1, 4, NB, 16), lambda i: (i, 0, 0, 0))
    out = pl.pallas_call(
        _attn_body,
        grid=(BH,),
        in_specs=[spec, spec, spec, mspec],
        out_specs=spec,
        out_shape=jax.ShapeDtypeStruct((BH, L, D), jnp.float32),
    )(qf, kf, vf, maskg.reshape(BH, 4, NB, 16))
    return out.reshape(B, H, L, D)


# speedup vs baseline: 1.2316x; 1.0755x over previous
import functools

import jax
import jax.numpy as jnp
from jax import lax
from jax.experimental import pallas as pl
from jax.experimental.pallas import tpu as pltpu
from jax.experimental.pallas import tpu_sc as plsc

NB = 64
BLK = 64
TOPK = 6
NW = 32
NG = 96
NEG = float("-inf")


def _scores_body(pq_ref, pk_ref, s_ref):
    s = lax.dot_general(
        pq_ref[...].astype(jnp.bfloat16), pk_ref[...].astype(jnp.bfloat16),
        (((2,), (2,)), ((0,), (0,))),
        preferred_element_type=jnp.float32)
    s_ref[...] = jnp.stack(
        [jnp.swapaxes(s[:, 16 * gi:16 * (gi + 1), :], 1, 2)
         for gi in range(4)], axis=1)


def _tree_max(vals):
    vals = list(vals)
    while len(vals) > 1:
        vals = [jnp.maximum(vals[i], vals[i + 1])
                for i in range(0, len(vals) - 1, 2)] + (
                    [vals[-1]] if len(vals) % 2 else [])
    return vals[0]


def _make_topk_kernel():
    mesh = plsc.VectorSubcoreMesh(core_axis_name="c", subcore_axis_name="s")

    @functools.partial(
        pl.kernel, mesh=mesh,
        out_type=jax.ShapeDtypeStruct((NG, NB, 16), jnp.float32),
        compiler_params=pltpu.CompilerParams(use_tc_tiling_on_sc=True),
        scratch_types=[
            pltpu.VMEM((NB, 16), jnp.float32),
            pltpu.VMEM((NB, 16), jnp.float32),
        ],
    )
    def topk_mask(scg_hbm, maskg_hbm, sc_v, mk_v):
        wid = lax.axis_index("s") * 2 + lax.axis_index("c")
        negv = jnp.full((16,), NEG, jnp.float32)
        onev = jnp.full((16,), 1.0, jnp.float32)
        zerov = jnp.full((16,), 0.0, jnp.float32)
        for t in range(NG // NW):
            g = wid + NW * t
            pltpu.sync_copy(scg_hbm.at[g], sc_v)
            cur = [sc_v[p, :] for p in range(NB)]
            for _ in range(TOPK - 1):
                thr = _tree_max(cur)
                cur = [jnp.where(c >= thr, negv, c) for c in cur]
            thr = _tree_max(cur)
            for p in range(NB):
                sel = (cur[p] >= thr) | (cur[p] == negv)
                mk_v[p, :] = jnp.where(sel, onev, zerov)
            pltpu.sync_copy(mk_v, maskg_hbm.at[g])

    return topk_mask


def _softmax_bf16(x):
    xb = x.astype(jnp.bfloat16).astype(jnp.float32)
    m = jnp.max(xb, axis=-1, keepdims=True)
    e = jnp.exp(xb - m)
    return e / jnp.sum(e, axis=-1, keepdims=True)


def _attn_body(q_ref, k_ref, v_ref, mask_ref, o_ref):
    qf = q_ref[0]
    kf = k_ref[0]
    vf = v_ref[0]
    mg = mask_ref[0]
    mask = jnp.concatenate(
        [jnp.swapaxes(mg[gi], 0, 1) for gi in range(4)],
        axis=0)

    cq = _softmax_bf16(qf)
    ck = _softmax_bf16(kf)
    vb = vf.astype(jnp.bfloat16).astype(jnp.float32)

    ck3 = ck.reshape(NB, BLK, 64)
    vb3 = vb.reshape(NB, BLK, 64)

    M = lax.dot_general(
        ck3, vb3, (((1,), (1,)), ((0,), (0,))),
        preferred_element_type=jnp.float32)
    ksum = jnp.sum(ck3, axis=1)

    Msum = lax.dot_general(
        mask, M, (((1,), (0,)), ((), ())),
        preferred_element_type=jnp.float32)
    dsum = lax.dot_general(
        mask, ksum, (((1,), (0,)), ((), ())),
        preferred_element_type=jnp.float32)

    cq3 = cq.reshape(NB, BLK, 64)
    num = lax.dot_general(
        cq3, Msum, (((2,), (1,)), ((0,), (0,))),
        preferred_element_type=jnp.float32)
    den = lax.dot_general(
        cq3, dsum, (((2,), (1,)), ((0,), (0,))),
        preferred_element_type=jnp.float32)

    o = num / (den[..., None] + 1e-6)
    o_ref[0] = o.reshape(NB * BLK, 64)


@jax.jit
def kernel(q, k, v):
    B, H, L, D = q.shape
    BH = B * H
    pq = q.reshape(B, H, NB, BLK, D).mean(axis=3).reshape(BH, NB, D)
    pk = k.reshape(B, H, NB, BLK, D).mean(axis=3).reshape(BH, NB, D)

    scg = pl.pallas_call(
        _scores_body,
        out_shape=jax.ShapeDtypeStruct((BH, 4, NB, 16), jnp.float32),
    )(pq, pk)

    maskg = _make_topk_kernel()(scg.reshape(NG, NB, 16))

    qf = q.reshape(BH, L, D)
    kf = k.reshape(BH, L, D)
    vf = v.reshape(BH, L, D)
    spec = pl.BlockSpec((1, L, D), lambda i: (i, 0, 0))
    mspec = pl.BlockSpec((1, 4, NB, 16), lambda i: (i, 0, 0, 0))
    out = pl.pallas_call(
        _attn_body,
        grid=(BH,),
        in_specs=[spec, spec, spec, mspec],
        out_specs=spec,
        out_shape=jax.ShapeDtypeStruct((BH, L, D), jnp.float32),
    )(qf, kf, vf, maskg.reshape(BH, 4, NB, 16))
    return out.reshape(B, H, L, D)
